# Initial kernel scaffold; baseline (speedup 1.0000x reference)
import jax
import jax.numpy as jnp
from jax.experimental import pallas as pl


def kernel(head_x, head_edge_index, head_edge_attr, head_batch, head_line_graph_edge_index, tail_x, tail_edge_index, tail_edge_attr, tail_batch, tail_line_graph_edge_index, relations, params):
    raise NotImplementedError("write your pallas kernel here")


# speedup vs baseline: 1.0501x; 1.0501x over previous
import functools

import jax
import jax.numpy as jnp
from jax import lax
from jax.experimental import pallas as pl
from jax.experimental.pallas import tpu as pltpu

IN_DIM = 128
HIDDEN = 128
KGE = 128
N_GRAPHS = 512
N_ITER = 2
N_BLOCKS = 3


def _mm_prelu_stats_body(x_ref, w_ref, b_ref, p_ref, inv_ref, y_ref, s1_ref, s2_ref):
    i = pl.program_id(0)
    x = x_ref[...]
    inv = inv_ref[...]
    y = jnp.dot(x, w_ref[...], preferred_element_type=jnp.float32)
    y = y * inv + b_ref[...]
    p = p_ref[0, 0]
    y = jnp.where(y >= 0.0, y, p * y)
    y_ref[...] = y

    @pl.when(i == 0)
    def _init():
        s1_ref[...] = jnp.zeros_like(s1_ref)
        s2_ref[...] = jnp.zeros_like(s2_ref)

    s1_ref[...] += jnp.sum(y, axis=0, keepdims=True)
    s2_ref[...] += jnp.sum(y * y, axis=0, keepdims=True)


def _mm_prelu_stats(x, inv, W, b, p, bm):
    E, K = x.shape
    N = W.shape[0]
    grid = E // bm
    y, s1, s2 = pl.pallas_call(
        _mm_prelu_stats_body,
        grid=(grid,),
        in_specs=[
            pl.BlockSpec((bm, K), lambda i: (i, 0)),
            pl.BlockSpec((K, N), lambda i: (0, 0)),
            pl.BlockSpec((1, N), lambda i: (0, 0)),
            pl.BlockSpec((1, 1), lambda i: (0, 0), memory_space=pltpu.SMEM),
            pl.BlockSpec((bm, 1), lambda i: (i, 0)),
        ],
        out_specs=[
            pl.BlockSpec((bm, N), lambda i: (i, 0)),
            pl.BlockSpec((1, N), lambda i: (0, 0)),
            pl.BlockSpec((1, N), lambda i: (0, 0)),
        ],
        out_shape=[
            jax.ShapeDtypeStruct((E, N), jnp.float32),
            jax.ShapeDtypeStruct((1, N), jnp.float32),
            jax.ShapeDtypeStruct((1, N), jnp.float32),
        ],
    )(x, W.T, b[None, :], p[None, None], inv)
    return y, s1[0], s2[0]


def _gru_body(u_ref, h_ref, inv_ref, wih_ref, whh_ref, bih_ref, bhh_ref, o_ref):
    x = u_ref[...] * inv_ref[...]
    h = h_ref[...]
    gi = jnp.dot(x, wih_ref[...], preferred_element_type=jnp.float32) + bih_ref[...]
    gh = jnp.dot(h, whh_ref[...], preferred_element_type=jnp.float32) + bhh_ref[...]
    H = h.shape[1]
    ir, iz, inn = gi[:, :H], gi[:, H:2 * H], gi[:, 2 * H:]
    hr, hz, hn = gh[:, :H], gh[:, H:2 * H], gh[:, 2 * H:]
    r = jax.nn.sigmoid(ir + hr)
    z = jax.nn.sigmoid(iz + hz)
    n = jnp.tanh(inn + r * hn)
    o_ref[...] = (1.0 - z) * n + z * h


def _gru(node_sum, inv, h, p, bm=2000):
    M, H = h.shape
    grid = M // bm
    return pl.pallas_call(
        _gru_body,
        grid=(grid,),
        in_specs=[
            pl.BlockSpec((bm, H), lambda i: (i, 0)),
            pl.BlockSpec((bm, H), lambda i: (i, 0)),
            pl.BlockSpec((bm, 1), lambda i: (i, 0)),
            pl.BlockSpec((H, 3 * H), lambda i: (0, 0)),
            pl.BlockSpec((H, 3 * H), lambda i: (0, 0)),
            pl.BlockSpec((1, 3 * H), lambda i: (0, 0)),
            pl.BlockSpec((1, 3 * H), lambda i: (0, 0)),
        ],
        out_specs=pl.BlockSpec((bm, H), lambda i: (i, 0)),
        out_shape=jax.ShapeDtypeStruct((M, H), jnp.float32),
    )(node_sum, h, inv, p['W_ih'].T, p['W_hh'].T, p['b_ih'][None, :], p['b_hh'][None, :])


def _linear(x, W, b=None):
    y = x @ W.T
    if b is not None:
        y = y + b
    return y


def _bn(x, g, b, eps=1e-5):
    mu = jnp.mean(x, axis=0)
    var = jnp.var(x, axis=0)
    return g * (x - mu) / jnp.sqrt(var + eps) + b


def _prelu(x, p):
    return jnp.where(x >= 0, x, p * x)


def _scatter_mean(v, idx, size):
    s = jax.ops.segment_sum(v, idx, num_segments=size)
    c = jax.ops.segment_sum(jnp.ones((v.shape[0],), v.dtype), idx, num_segments=size)
    return s / jnp.maximum(c, 1.0)[:, None]


def _inv_counts(idx, size):
    c = jax.ops.segment_sum(jnp.ones(idx.shape, jnp.float32), idx, num_segments=size)
    return (1.0 / jnp.maximum(c, 1.0))[:, None], c


def _gnp_block(p, x, edge_index, edge_attr, batch, lg, pre):
    h = x
    cur = x
    src, dst = edge_index[0], edge_index[1]
    lsrc, ldst = lg[0], lg[1]
    inv_lg, _ = pre['inv_lg'], None
    inv_dst = pre['inv_dst']
    E = edge_attr.shape[0]
    for _ in range(N_ITER):
        fused = edge_attr + (cur[src] + cur[dst]) / 2.0
        msgs = fused[lsrc]
        agg_sum = jax.ops.segment_sum(msgs, ldst, num_segments=E)
        y, s1, s2 = _mm_prelu_stats(agg_sum, inv_lg, p['e_W'], p['e_b'], p['e_p'], bm=1600)
        mu = s1 / E
        var = s2 / E - mu * mu
        scale = p['e_g'] / jnp.sqrt(var + 1e-5)
        shift = p['e_bb'] - scale * mu
        fused = fused + y * scale + shift
        node_sum = jax.ops.segment_sum(fused, dst, num_segments=cur.shape[0])
        h = _gru(node_sum, inv_dst, h, p)
        cur = h
    graph_sum = jax.ops.segment_sum(cur, batch, num_segments=N_GRAPHS)
    graph_repr = graph_sum * pre['inv_batch']
    a = _prelu(_linear(graph_repr, p['a_W1'], p['a_b1']), p['a_p'])
    a = jax.nn.sigmoid(_linear(a, p['a_W2'], p['a_b2']))
    graph_emb = _linear(graph_sum * a, p['r_W'], p['r_b'])
    return cur, graph_emb


def kernel(head_x, head_edge_index, head_edge_attr, head_batch, head_line_graph_edge_index, tail_x, tail_edge_index, tail_edge_attr, tail_batch, tail_line_graph_edge_index, relations, params):
    pp = params['pre']

    def preproc(x):
        x = _linear(x, pp['W1'], pp['b1'])
        x = _bn(x, pp['g1'], pp['bb1'])
        x = _prelu(x, pp['p'])
        x = _linear(x, pp['W2'], pp['b2'])
        x = _bn(x, pp['g2'], pp['bb2'])
        return x

    N = head_x.shape[0]
    E = head_edge_attr.shape[0]

    def make_pre(lg, edge_index, batch):
        inv_lg, _ = _inv_counts(lg[1], E)
        inv_dst, _ = _inv_counts(edge_index[1], N)
        inv_batch, _ = _inv_counts(batch, N_GRAPHS)
        return {'inv_lg': inv_lg, 'inv_dst': inv_dst, 'inv_batch': inv_batch}

    hpre = make_pre(head_line_graph_edge_index, head_edge_index, head_batch)
    tpre = make_pre(tail_line_graph_edge_index, tail_edge_index, tail_batch)

    hx = preproc(head_x)
    tx = preproc(tail_x)
    he = head_edge_attr @ params['edge_W'].T
    te = tail_edge_attr @ params['edge_W'].T
    head_reps = []
    tail_reps = []
    for bi in range(N_BLOCKS):
        bp = params['blocks'][bi]
        uhx, hge = _gnp_block(bp, hx, head_edge_index, he, head_batch, head_line_graph_edge_index, hpre)
        utx, tge = _gnp_block(bp, tx, tail_edge_index, te, tail_batch, tail_line_graph_edge_index, tpre)
        head_reps.append(hge)
        tail_reps.append(tge)
        hx = (hx + uhx) / 2.0
        tx = (tx + utx) / 2.0
    hm = jnp.stack(head_reps, axis=1)
    tm = jnp.stack(tail_reps, axis=1)
    ca = params['coatt']
    keys = hm @ ca['w_k']
    queries = tm @ ca['w_q']
    e = queries[:, None, :, :] + keys[:, :, None, :] + ca['bias']
    att = jnp.tanh(e) @ ca['a']
    hw = att[:, :, 0][..., None]
    tw = att[:, :, 1][..., None]
    fh = jnp.sum(hm * hw, axis=1)
    ft = jnp.sum(tm * tw, axis=1)
    M = params['rel_emb'][relations].reshape(-1, KGE, KGE)
    scores = jnp.einsum('bi,bij,bj->b', fh, M, ft)
    comb = jnp.concatenate([hm.mean(axis=1), tm.mean(axis=1)], axis=1)
    u = params['unc']
    unc = jax.nn.softplus(_linear(_prelu(_linear(comb, u['W1'], u['b1']), u['p']), u['W2'], u['b2']))
    return scores, unc.squeeze(-1)
